# baseline (speedup 1.0000x reference)
import functools

import jax
import jax.numpy as jnp
from jax import lax
from jax.experimental import pallas as pl
from jax.experimental.pallas import tpu as pltpu
from jax.experimental.pallas import tpu_sc as plsc

B = 1024
N = 100000
RS = 256
NC = 2
NS = 16
NW = NC * NS
ROWS = RS // NW
L = 16
CH = 20000
NCH = N // CH
NV = CH // L
U = 10
BIG = 2**30
NINF = float("-inf")

_mesh = plsc.VectorSubcoreMesh(core_axis_name="c", subcore_axis_name="s")


@functools.partial(
    pl.kernel,
    mesh=_mesh,
    out_type=jax.ShapeDtypeStruct((NW * L,), jnp.int32),
    scratch_types=[
        pltpu.VMEM((CH,), jnp.float32),
        pltpu.VMEM((CH,), jnp.float32),
        pltpu.VMEM((CH,), jnp.float32),
        pltpu.VMEM((CH,), jnp.float32),
        pltpu.VMEM((L,), jnp.float32),
        pltpu.VMEM((L,), jnp.int32),
        pltpu.SemaphoreType.DMA,
        pltpu.SemaphoreType.DMA,
        pltpu.SemaphoreType.DMA,
        pltpu.SemaphoreType.DMA,
    ],
)
def _sc_precision(yhat_hbm, y_hbm, out_hbm, by0, by1, bh0, bh1, vbuf, obuf,
                  sy0, sy1, sh0, sh1):
    wid = lax.axis_index("s") * NC + lax.axis_index("c")
    iota = lax.iota(jnp.int32, L)
    by = (by0, by1)
    bh = (bh0, bh1)
    sy = (sy0, sy1)
    sh = (sh0, sh1)
    one = jnp.full((L,), 1, jnp.int32)
    zero = jnp.zeros((L,), jnp.int32)

    obuf[...] = zero

    def row_body(j, carry):
        r = wid * ROWS + j
        base = r * N

        cp_y = [pltpu.async_copy(y_hbm.at[pl.ds(base + c * CH, CH)],
                                 by[c % 2], sy[c % 2]) for c in range(2)]
        cp_h = [pltpu.async_copy(yhat_hbm.at[pl.ds(base + c * CH, CH)],
                                 bh[c % 2], sh[c % 2]) for c in range(2)]

        bv0 = jnp.full((L,), -jnp.inf, jnp.float32)
        bv1 = bv0
        bi0 = jnp.full((L,), BIG, jnp.int32)
        bi1 = bi0
        for c in range(NCH):
            cp_y[c].wait()
            buf = by[c % 2]

            @plsc.parallel_loop(0, NV, U, carry=(bv0, bi0, bv1, bi1))
            def p1_out(i, acc, _buf=buf, _c=c):
                bv0, bi0, bv1, bi1 = acc
                for u in range(U):
                    vec = _buf[pl.ds((i + u) * L, L)]
                    gidx = jnp.full((L,), _c * NV + i + u, jnp.int32)
                    if u % 2 == 0:
                        g = vec > bv0
                        bv0 = jnp.where(g, vec, bv0)
                        bi0 = jnp.where(g, gidx, bi0)
                    else:
                        g = vec > bv1
                        bv1 = jnp.where(g, vec, bv1)
                        bi1 = jnp.where(g, gidx, bi1)
                return bv0, bi0, bv1, bi1

            bv0, bi0, bv1, bi1 = p1_out
            if c + 2 < NCH:
                cp_y.append(pltpu.async_copy(
                    y_hbm.at[pl.ds(base + (c + 2) * CH, CH)],
                    by[c % 2], sy[c % 2]))

        take = (bv1 > bv0) | ((bv1 == bv0) & (bi1 < bi0))
        bv = jnp.where(take, bv1, bv0)
        bi = jnp.where(take, bi1, bi0)
        ei = bi * L + iota
        for sh_ in (8, 4, 2, 1):
            perm = iota ^ sh_
            ov = bv[perm]
            oi = ei[perm]
            take = (ov > bv) | ((ov == bv) & (oi < ei))
            bv = jnp.where(take, ov, bv)
            ei = jnp.where(take, oi, ei)
        t = ei[0]
        tvec = ei
        tv = t // L

        pltpu.sync_copy(yhat_hbm.at[pl.ds(base + tv * L, L)], vbuf)
        vwin = vbuf[...]
        vv = jnp.where(iota == (t - tv * L), vwin, -jnp.inf)
        for sh_ in (8, 4, 2, 1):
            vv = jnp.maximum(vv, vv[iota ^ sh_])
        v = vv

        g0 = zero
        e0 = zero
        g1 = zero
        e1 = zero
        for c in range(NCH):
            cp_h[c].wait()
            buf = bh[c % 2]
            nA = jnp.clip(tv - c * NV, 0, NV)
            nAq = (nA // U) * U
            mid_hi = jnp.minimum(nAq + U, NV)

            @plsc.parallel_loop(0, nAq, U, carry=(g0, e0, g1, e1))
            def pA_out(i, acc, _buf=buf):
                g0, e0, g1, e1 = acc
                for u in range(U):
                    vec = _buf[pl.ds((i + u) * L, L)]
                    if u % 2 == 0:
                        g0 = g0 + jnp.where(vec > v, one, zero)
                        e0 = e0 + jnp.where(vec == v, one, zero)
                    else:
                        g1 = g1 + jnp.where(vec > v, one, zero)
                        e1 = e1 + jnp.where(vec == v, one, zero)
                return g0, e0, g1, e1

            g0, e0, g1, e1 = pA_out

            @plsc.parallel_loop(nAq, mid_hi, 1, carry=(g0, e0))
            def pM_out(i, acc, _buf=buf, _c=c):
                g0, e0 = acc
                vec = _buf[pl.ds(i * L, L)]
                ivec = jnp.full((L,), (_c * NV + i) * L, jnp.int32) + iota
                g0 = g0 + jnp.where(vec > v, one, zero)
                e0 = e0 + jnp.where((vec == v) & (ivec < tvec), one, zero)
                return g0, e0

            g0, e0 = pM_out

            @plsc.parallel_loop(mid_hi, NV, U, carry=(g0, g1))
            def pB_out(i, acc, _buf=buf):
                g0, g1 = acc
                for u in range(U):
                    vec = _buf[pl.ds((i + u) * L, L)]
                    if u % 2 == 0:
                        g0 = g0 + jnp.where(vec > v, one, zero)
                    else:
                        g1 = g1 + jnp.where(vec > v, one, zero)
                return g0, g1

            g0, g1 = pB_out
            if c + 2 < NCH:
                cp_h.append(pltpu.async_copy(
                    yhat_hbm.at[pl.ds(base + (c + 2) * CH, CH)],
                    bh[c % 2], sh[c % 2]))

        rk = g0 + g1 + e0 + e1
        for sh_ in (8, 4, 2, 1):
            rk = rk + rk[iota ^ sh_]
        incr = jnp.where(iota == 0, jnp.where(rk < 1, one, zero),
                         jnp.where(iota == 1, jnp.where(rk < 5, one, zero),
                                   jnp.where(iota == 2,
                                             jnp.where(rk < 10, one, zero),
                                             zero)))
        obuf[...] = obuf[...] + incr
        return carry

    lax.fori_loop(0, ROWS, row_body, jnp.int32(0))
    pltpu.sync_copy(obuf, out_hbm.at[pl.ds(wid * L, L)])


CK = 2000
NST = N // CK
WB = 256
NWB = (B - RS) // WB


def _k1_body(y_ref, h_ref, of_ref, oi_ref):
    i = pl.program_id(0)
    yv = y_ref[...]
    hvk = h_ref[...]
    io0 = lax.broadcasted_iota(jnp.int32, (CK, WB), 0)
    cm = jnp.max(yv, axis=0)
    carg = jnp.min(jnp.where(yv == cm[None, :], io0, BIG), axis=0)
    hc = jnp.max(jnp.where(io0 == carg[None, :], hvk, NINF), axis=0)

    @pl.when(i == 0)
    def _():
        of_ref[0, :] = cm
        of_ref[1, :] = hc
        oi_ref[0, :] = carg

    @pl.when(i > 0)
    def _():
        bm = of_ref[0, :]
        better = cm > bm
        of_ref[0, :] = jnp.where(better, cm, bm)
        of_ref[1, :] = jnp.where(better, hc, of_ref[1, :])
        oi_ref[0, :] = jnp.where(better, i * CK + carg, oi_ref[0, :])


_tc_k1 = pl.pallas_call(
    _k1_body,
    grid=(NST, NWB),
    in_specs=[pl.BlockSpec((CK, WB), lambda i, j: (i, j + RS // WB)),
              pl.BlockSpec((CK, WB), lambda i, j: (i, j + RS // WB))],
    out_specs=[pl.BlockSpec((8, WB), lambda i, j: (0, j)),
               pl.BlockSpec((8, WB), lambda i, j: (0, j))],
    out_shape=[jax.ShapeDtypeStruct((8, B - RS), jnp.float32),
               jax.ShapeDtypeStruct((8, B - RS), jnp.int32)],
)


def _k2_body(h_ref, tv_ref, vv_ref, acc_ref, out_ref):
    i = pl.program_id(0)
    j = pl.program_id(1)
    hvk = h_ref[...]
    tv = tv_ref[0, :]
    vv = vv_ref[1, :]
    io0 = lax.broadcasted_iota(jnp.int32, (CK, WB), 0)
    gt_c = jnp.sum((hvk > vv[None, :]).astype(jnp.int32), axis=0)
    gidx = i * CK + io0
    eq_c = jnp.sum(((hvk == vv[None, :]) & (gidx < tv[None, :]))
                   .astype(jnp.int32), axis=0)

    @pl.when(i == 0)
    def _():
        acc_ref[0, :] = gt_c
        acc_ref[1, :] = eq_c

    @pl.when(i > 0)
    def _():
        acc_ref[0, :] = acc_ref[0, :] + gt_c
        acc_ref[1, :] = acc_ref[1, :] + eq_c

    @pl.when(i == NST - 1)
    def _():
        rank = acc_ref[0, :] + acc_ref[1, :]
        c1 = jnp.sum((rank < 1).astype(jnp.int32))
        c5 = jnp.sum((rank < 5).astype(jnp.int32))
        c10 = jnp.sum((rank < 10).astype(jnp.int32))
        io = lax.broadcasted_iota(jnp.int32, (1, 1, 128), 2)
        vec = jnp.where(io == 0, c1,
                        jnp.where(io == 1, c5,
                                  jnp.where(io == 2, c10, 0)))

        @pl.when(j == 0)
        def _():
            out_ref[...] = vec

        @pl.when(j > 0)
        def _():
            out_ref[...] = out_ref[...] + vec


_tc_k2 = pl.pallas_call(
    _k2_body,
    grid=(NST, NWB),
    in_specs=[pl.BlockSpec((CK, WB), lambda i, j: (i, j + RS // WB)),
              pl.BlockSpec((8, WB), lambda i, j: (0, j)),
              pl.BlockSpec((8, WB), lambda i, j: (0, j))],
    out_specs=[pl.BlockSpec((8, WB), lambda i, j: (0, j)),
               pl.BlockSpec((1, 1, 128), lambda i, j: (0, 0, 0))],
    out_shape=[jax.ShapeDtypeStruct((8, B - RS), jnp.int32),
               jax.ShapeDtypeStruct((1, 1, 128), jnp.int32)],
)


def kernel(yhat, y):
    sc_counts = _sc_precision(yhat[:RS].reshape(-1), y[:RS].reshape(-1))
    yT = y.T
    hT = yhat.T
    fo, iu = _tc_k1(yT, hT)
    _, tc_counts = _tc_k2(hT, iu, fo)
    per_k = (sc_counts.reshape(NW, L)[:, :3].sum(axis=0)
             + tc_counts[0, 0, :3])
    return per_k.astype(jnp.float32) / jnp.float32(B)


# speedup vs baseline: 1.3131x; 1.3131x over previous
import functools

import jax
import jax.numpy as jnp
from jax import lax
from jax.experimental import pallas as pl
from jax.experimental.pallas import tpu as pltpu
from jax.experimental.pallas import tpu_sc as plsc

B = 1024
N = 100000
RS = 256
NC = 2
NS = 16
NW = NC * NS
NB = RS // (NW * 8)
L = 16
CW = 1920
NVR = CW // L
U = 12
NFC = 52
TAILC = NFC * CW
TAILW = N - TAILC
NVT = TAILW // L
BIG = 2**30
NINF = float("-inf")

_mesh = plsc.VectorSubcoreMesh(core_axis_name="c", subcore_axis_name="s")


@functools.partial(
    pl.kernel,
    mesh=_mesh,
    compiler_params=pltpu.CompilerParams(skip_device_barrier=True),
    out_type=jax.ShapeDtypeStruct((NW * L,), jnp.int32),
    scratch_types=[
        pltpu.VMEM((8, CW), jnp.float32),
        pltpu.VMEM((8, CW), jnp.float32),
        pltpu.VMEM((8, CW), jnp.float32),
        pltpu.VMEM((8, CW), jnp.float32),
        pltpu.VMEM((8, TAILW), jnp.float32),
        pltpu.VMEM((8, TAILW), jnp.float32),
        pltpu.VMEM((8, 8, 128), jnp.float32),
        pltpu.VMEM((8 * L,), jnp.float32),
        pltpu.VMEM((8 * L,), jnp.float32),
        pltpu.VMEM((8 * L,), jnp.int32),
        pltpu.VMEM((8 * L,), jnp.int32),
        pltpu.VMEM((8 * L,), jnp.int32),
        pltpu.VMEM((8 * L,), jnp.float32),
        pltpu.VMEM((8 * L,), jnp.int32),
        pltpu.VMEM((8 * L,), jnp.int32),
        pltpu.VMEM((L,), jnp.int32),
        pltpu.SemaphoreType.DMA,
        pltpu.SemaphoreType.DMA,
        pltpu.SemaphoreType.DMA,
        pltpu.SemaphoreType.DMA,
        pltpu.SemaphoreType.DMA,
        pltpu.SemaphoreType.DMA,
        pltpu.SemaphoreType.DMA,
    ],
)
def _sc_precision(yhat_hbm, y_hbm, out_hbm,
                  yb0, yb1, hb0, hb1, ytl, htl, wbuf,
                  bvr0, bvr1, bir0, bir1, tvr, vvr, ger, eer, obuf,
                  sy0, sy1, sh0, sh1, sty, sth, sw):
    wid = lax.axis_index("s") * NC + lax.axis_index("c")
    iota = lax.iota(jnp.int32, L)
    one = jnp.full((L,), 1, jnp.int32)
    zero = jnp.zeros((L,), jnp.int32)
    ninf = jnp.full((L,), NINF, jnp.float32)
    big = jnp.full((L,), BIG, jnp.int32)
    yb = (yb0, yb1)
    hb = (hb0, hb1)
    sy = (sy0, sy1)
    sh = (sh0, sh1)

    obuf[...] = zero

    def p1_chunk(buf, cv0, k):
        s = pl.ds(k * L, L)
        bv0 = bvr0[s]
        bv1 = bvr1[s]
        bi0 = bir0[s]
        bi1 = bir1[s]

        @plsc.parallel_loop(0, NVR, U, carry=(bv0, bi0, bv1, bi1))
        def p1_out(i, acc):
            bv0, bi0, bv1, bi1 = acc
            for u in range(U):
                vec = buf[k, pl.ds((i + u) * L, L)]
                gidx = jnp.full((L,), cv0 + i + u, jnp.int32)
                if u % 2 == 0:
                    g = vec > bv0
                    bv0 = jnp.where(g, vec, bv0)
                    bi0 = jnp.where(g, gidx, bi0)
                else:
                    g = vec > bv1
                    bv1 = jnp.where(g, vec, bv1)
                    bi1 = jnp.where(g, gidx, bi1)
            return bv0, bi0, bv1, bi1

        bv0, bi0, bv1, bi1 = p1_out
        bvr0[s] = bv0
        bvr1[s] = bv1
        bir0[s] = bi0
        bir1[s] = bi1

    def p2_chunk(buf, c0, k):
        s = pl.ds(k * L, L)
        g = ger[s]
        e = eer[s]
        v = vvr[s]
        tvec = tvr[s]
        t_s = tvec[0]
        tv = t_s // L
        cv0 = c0 // L
        nA = jnp.clip(tv - cv0, 0, NVR)
        nAq = (nA // U) * U
        mid_hi = jnp.minimum(nAq + U, NVR)

        @plsc.parallel_loop(0, nAq, U, carry=(g, e))
        def pA_out(i, acc):
            g, e = acc
            for u in range(U):
                vec = buf[k, pl.ds((i + u) * L, L)]
                g = g + jnp.where(vec > v, one, zero)
                e = e + jnp.where(vec == v, one, zero)
            return g, e

        g, e = pA_out

        @plsc.parallel_loop(nAq, mid_hi, 1, carry=(g, e))
        def pM_out(i, acc):
            g, e = acc
            vec = buf[k, pl.ds(i * L, L)]
            ivec = jnp.full((L,), c0 + i * L, jnp.int32) + iota
            g = g + jnp.where(vec > v, one, zero)
            e = e + jnp.where((vec == v) & (ivec < tvec), one, zero)
            return g, e

        g, e = pM_out

        @plsc.parallel_loop(mid_hi, NVR, U, carry=g)
        def pB_out(i, acc):
            g = acc
            for u in range(U):
                vec = buf[k, pl.ds((i + u) * L, L)]
                g = g + jnp.where(vec > v, one, zero)
            return g

        ger[s] = pB_out
        eer[s] = e

    def block_body(tb, carry):
        r8 = pl.multiple_of((wid * NB + tb) * 8, 8)

        for k in range(8):
            s = pl.ds(k * L, L)
            bvr0[s] = ninf
            bvr1[s] = ninf
            bir0[s] = big
            bir1[s] = big

        pltpu.async_copy(y_hbm.at[pl.ds(r8, 8), pl.ds(0, CW)], yb0, sy0)
        pltpu.async_copy(y_hbm.at[pl.ds(r8, 8), pl.ds(CW, CW)], yb1, sy1)
        pltpu.async_copy(yhat_hbm.at[pl.ds(r8, 8), pl.ds(0, CW)], hb0, sh0)
        pltpu.async_copy(yhat_hbm.at[pl.ds(r8, 8), pl.ds(CW, CW)], hb1, sh1)
        pltpu.async_copy(y_hbm.at[pl.ds(r8, 8), pl.ds(TAILC, TAILW)],
                         ytl, sty)
        pltpu.async_copy(yhat_hbm.at[pl.ds(r8, 8), pl.ds(TAILC, TAILW)],
                         htl, sth)

        def p1_body(i, car):
            c0 = i * (2 * CW)
            for p in range(2):
                cc = c0 + p * CW
                pltpu.make_async_copy(
                    y_hbm.at[pl.ds(r8, 8),
                             pl.ds(pl.multiple_of(cc, 128), CW)],
                    yb[p], sy[p]).wait()
                for k in range(8):
                    p1_chunk(yb[p], cc // L, k)
                nxt = jnp.minimum(2 * i + 2 + p, NFC - 2 + p) * CW
                pltpu.async_copy(
                    y_hbm.at[pl.ds(r8, 8),
                             pl.ds(pl.multiple_of(nxt, 128), CW)],
                    yb[p], sy[p])
            return car

        lax.fori_loop(0, NFC // 2, p1_body, jnp.int32(0))
        for p in range(2):
            pltpu.make_async_copy(
                y_hbm.at[pl.ds(r8, 8), pl.ds((NFC - 2 + p) * CW, CW)],
                yb[p], sy[p]).wait()

        pltpu.make_async_copy(
            y_hbm.at[pl.ds(r8, 8), pl.ds(TAILC, TAILW)], ytl, sty).wait()
        for k in range(8):
            s = pl.ds(k * L, L)
            bv0 = bvr0[s]
            bi0 = bir0[s]
            for u in range(NVT):
                vec = ytl[k, pl.ds(u * L, L)]
                gidx = jnp.full((L,), TAILC // L + u, jnp.int32)
                g = vec > bv0
                bv0 = jnp.where(g, vec, bv0)
                bi0 = jnp.where(g, gidx, bi0)
            bvr0[s] = bv0
            bir0[s] = bi0

        wcps = []
        for k in range(8):
            s = pl.ds(k * L, L)
            bv0 = bvr0[s]
            bv1 = bvr1[s]
            bi0 = bir0[s]
            bi1 = bir1[s]
            take = (bv1 > bv0) | ((bv1 == bv0) & (bi1 < bi0))
            bv = jnp.where(take, bv1, bv0)
            bi = jnp.where(take, bi1, bi0)
            ei = bi * L + iota
            for sh_ in (8, 4, 2, 1):
                perm = iota ^ sh_
                ov = bv[perm]
                oi = ei[perm]
                take = (ov > bv) | ((ov == bv) & (oi < ei))
                bv = jnp.where(take, ov, bv)
                ei = jnp.where(take, oi, ei)
            tvr[s] = ei
            t_s = ei[0]
            wstart = jnp.minimum((t_s // 128) * 128, ((N // 128) - 2) * 128)
            wcps.append(pltpu.async_copy(
                yhat_hbm.at[pl.ds(r8, 8),
                            pl.ds(pl.multiple_of(wstart, 128), 128)],
                wbuf.at[k], sw))
        for cp in wcps:
            cp.wait()
        pltpu.make_async_copy(
            yhat_hbm.at[pl.ds(r8, 8), pl.ds(TAILC, TAILW)], htl, sth).wait()

        for k in range(8):
            s = pl.ds(k * L, L)
            tvec = tvr[s]
            t_s = tvec[0]
            wstart = jnp.minimum((t_s // 128) * 128, ((N // 128) - 2) * 128)
            w0 = ((t_s - wstart) // L) * L
            w0 = pl.multiple_of(jnp.clip(w0, 0, 128 - L), L)
            wv = wbuf[k, k, pl.ds(w0, L)]
            gcol_a = jnp.full((L,), wstart + w0, jnp.int32) + iota
            va = jnp.where(gcol_a == tvec, wv, NINF)
            w0b = pl.multiple_of(
                jnp.clip(((t_s - TAILC) // L) * L, 0, TAILW - L), L)
            tv2 = htl[k, pl.ds(w0b, L)]
            gcol_b = jnp.full((L,), TAILC + w0b, jnp.int32) + iota
            vv = jnp.maximum(va, jnp.where(gcol_b == tvec, tv2, NINF))
            for sh_ in (8, 4, 2, 1):
                vv = jnp.maximum(vv, vv[iota ^ sh_])
            vvr[s] = vv
            ger[s] = zero
            eer[s] = zero

        def p2_body(i, car):
            c0 = i * (2 * CW)
            for p in range(2):
                cc = c0 + p * CW
                pltpu.make_async_copy(
                    yhat_hbm.at[pl.ds(r8, 8),
                                pl.ds(pl.multiple_of(cc, 128), CW)],
                    hb[p], sh[p]).wait()
                for k in range(8):
                    p2_chunk(hb[p], cc, k)
                nxt = jnp.minimum(2 * i + 2 + p, NFC - 2 + p) * CW
                pltpu.async_copy(
                    yhat_hbm.at[pl.ds(r8, 8),
                                pl.ds(pl.multiple_of(nxt, 128), CW)],
                    hb[p], sh[p])
            return car

        lax.fori_loop(0, NFC // 2, p2_body, jnp.int32(0))
        for p in range(2):
            pltpu.make_async_copy(
                yhat_hbm.at[pl.ds(r8, 8), pl.ds((NFC - 2 + p) * CW, CW)],
                hb[p], sh[p]).wait()

        for k in range(8):
            s = pl.ds(k * L, L)
            g = ger[s]
            e = eer[s]
            v = vvr[s]
            tvec = tvr[s]
            for u in range(NVT):
                vec = htl[k, pl.ds(u * L, L)]
                ivec = jnp.full((L,), TAILC + u * L, jnp.int32) + iota
                g = g + jnp.where(vec > v, one, zero)
                e = e + jnp.where((vec == v) & (ivec < tvec), one, zero)
            rk = g + e
            for sh_ in (8, 4, 2, 1):
                rk = rk + rk[iota ^ sh_]
            incr = jnp.where(
                iota == 0, jnp.where(rk < 1, one, zero),
                jnp.where(iota == 1, jnp.where(rk < 5, one, zero),
                          jnp.where(iota == 2, jnp.where(rk < 10, one, zero),
                                    zero)))
            obuf[...] = obuf[...] + incr
        return carry

    lax.fori_loop(0, NB, block_body, jnp.int32(0))
    pltpu.sync_copy(obuf, out_hbm.at[pl.ds(wid * L, L)])


CK = 2000
NST = N // CK
WB = 256
NWB = (B - RS) // WB


def _k1_body(y_ref, h_ref, of_ref, oi_ref):
    i = pl.program_id(0)
    yv = y_ref[...]
    hvk = h_ref[...]
    io0 = lax.broadcasted_iota(jnp.int32, (CK, WB), 0)
    cm = jnp.max(yv, axis=0)
    carg = jnp.min(jnp.where(yv == cm[None, :], io0, BIG), axis=0)
    hc = jnp.max(jnp.where(io0 == carg[None, :], hvk, NINF), axis=0)

    @pl.when(i == 0)
    def _():
        of_ref[0, :] = cm
        of_ref[1, :] = hc
        oi_ref[0, :] = carg

    @pl.when(i > 0)
    def _():
        bm = of_ref[0, :]
        better = cm > bm
        of_ref[0, :] = jnp.where(better, cm, bm)
        of_ref[1, :] = jnp.where(better, hc, of_ref[1, :])
        oi_ref[0, :] = jnp.where(better, i * CK + carg, oi_ref[0, :])


_tc_k1 = pl.pallas_call(
    _k1_body,
    grid=(NST, NWB),
    in_specs=[pl.BlockSpec((CK, WB), lambda i, j: (i, j + RS // WB)),
              pl.BlockSpec((CK, WB), lambda i, j: (i, j + RS // WB))],
    out_specs=[pl.BlockSpec((8, WB), lambda i, j: (0, j)),
               pl.BlockSpec((8, WB), lambda i, j: (0, j))],
    out_shape=[jax.ShapeDtypeStruct((8, B - RS), jnp.float32),
               jax.ShapeDtypeStruct((8, B - RS), jnp.int32)],
)


def _k2_body(h_ref, tv_ref, vv_ref, acc_ref, out_ref):
    i = pl.program_id(0)
    j = pl.program_id(1)
    hvk = h_ref[...]
    tv = tv_ref[0, :]
    vv = vv_ref[1, :]
    io0 = lax.broadcasted_iota(jnp.int32, (CK, WB), 0)
    gt_c = jnp.sum((hvk > vv[None, :]).astype(jnp.int32), axis=0)
    gidx = i * CK + io0
    eq_c = jnp.sum(((hvk == vv[None, :]) & (gidx < tv[None, :]))
                   .astype(jnp.int32), axis=0)

    @pl.when(i == 0)
    def _():
        acc_ref[0, :] = gt_c
        acc_ref[1, :] = eq_c

    @pl.when(i > 0)
    def _():
        acc_ref[0, :] = acc_ref[0, :] + gt_c
        acc_ref[1, :] = acc_ref[1, :] + eq_c

    @pl.when(i == NST - 1)
    def _():
        rank = acc_ref[0, :] + acc_ref[1, :]
        c1 = jnp.sum((rank < 1).astype(jnp.int32))
        c5 = jnp.sum((rank < 5).astype(jnp.int32))
        c10 = jnp.sum((rank < 10).astype(jnp.int32))
        io = lax.broadcasted_iota(jnp.int32, (1, 1, 128), 2)
        vec = jnp.where(io == 0, c1,
                        jnp.where(io == 1, c5,
                                  jnp.where(io == 2, c10, 0)))

        @pl.when(j == 0)
        def _():
            out_ref[...] = vec

        @pl.when(j > 0)
        def _():
            out_ref[...] = out_ref[...] + vec


_tc_k2 = pl.pallas_call(
    _k2_body,
    grid=(NST, NWB),
    in_specs=[pl.BlockSpec((CK, WB), lambda i, j: (i, j + RS // WB)),
              pl.BlockSpec((8, WB), lambda i, j: (0, j)),
              pl.BlockSpec((8, WB), lambda i, j: (0, j))],
    out_specs=[pl.BlockSpec((8, WB), lambda i, j: (0, j)),
               pl.BlockSpec((1, 1, 128), lambda i, j: (0, 0, 0))],
    out_shape=[jax.ShapeDtypeStruct((8, B - RS), jnp.int32),
               jax.ShapeDtypeStruct((1, 1, 128), jnp.int32)],
)


def kernel(yhat, y):
    sc_counts = _sc_precision(yhat[:RS], y[:RS])
    yT = y.T
    hT = yhat.T
    fo, iu = _tc_k1(yT, hT)
    _, tc_counts = _tc_k2(hT, iu, fo)
    per_k = (sc_counts.reshape(NW, L)[:, :3].sum(axis=0)
             + tc_counts[0, 0, :3])
    return per_k.astype(jnp.float32) / jnp.float32(B)
